# baseline (speedup 1.0000x reference)
import jax
import jax.numpy as jnp
from jax.experimental import pallas as pl
from jax.experimental.pallas import tpu as pltpu

BN_EPS = 1e-5
NORM_EPS = 1e-12
BLOCK_N = 1024


def _netvlad_kernel(x_ref, cl_ref, cl2_ref, g_ref, b_ref, m_ref, v_ref,
                    out_ref, acc_ref, asum_ref):
    j = pl.program_id(1)
    nb = pl.num_programs(1)
    K = cl2_ref.shape[2]

    @pl.when(j == 0)
    def _():
        acc_ref[...] = jnp.zeros_like(acc_ref)
        asum_ref[...] = jnp.zeros_like(asum_ref)

    scale = g_ref[...] * jax.lax.rsqrt(v_ref[...] + BN_EPS)
    bias = b_ref[...] - m_ref[...] * scale
    xb = x_ref[0]
    logits = jnp.dot(xb, cl_ref[...] * scale,
                     preferred_element_type=jnp.float32) + bias
    mx = jnp.max(logits, axis=-1, keepdims=True)
    e = jnp.exp(logits - mx)
    s = jnp.sum(e, axis=-1, keepdims=True)
    a = e[:, :K] / s
    acc_ref[...] += jax.lax.dot_general(
        xb, a, (((0,), (0,)), ((), ())),
        preferred_element_type=jnp.float32)
    asum_ref[...] += jnp.sum(a, axis=0, keepdims=True)

    @pl.when(j == nb - 1)
    def _():
        vlad = acc_ref[...] - asum_ref[...] * cl2_ref[0]
        n1 = jnp.sqrt(jnp.sum(vlad * vlad, axis=0, keepdims=True))
        vlad = vlad / jnp.maximum(n1, NORM_EPS)
        n2 = jnp.sqrt(jnp.sum(vlad * vlad))
        vlad = vlad / jnp.maximum(n2, NORM_EPS)
        out_ref[0] = vlad


def kernel(x, clusters, clusters2, bn_gamma, bn_beta, bn_mean, bn_var):
    B, N, D = x.shape
    C = clusters.shape[1]
    K = clusters2.shape[2]
    nb = N // BLOCK_N

    out = pl.pallas_call(
        _netvlad_kernel,
        out_shape=jax.ShapeDtypeStruct((B, D, K), jnp.float32),
        grid=(B, nb),
        in_specs=[
            pl.BlockSpec((1, BLOCK_N, D), lambda b, j: (b, j, 0)),
            pl.BlockSpec((D, C), lambda b, j: (0, 0)),
            pl.BlockSpec((1, D, K), lambda b, j: (0, 0, 0)),
            pl.BlockSpec((1, C), lambda b, j: (0, 0)),
            pl.BlockSpec((1, C), lambda b, j: (0, 0)),
            pl.BlockSpec((1, C), lambda b, j: (0, 0)),
            pl.BlockSpec((1, C), lambda b, j: (0, 0)),
        ],
        out_specs=pl.BlockSpec((1, D, K), lambda b, j: (b, 0, 0)),
        scratch_shapes=[
            pltpu.VMEM((D, K), jnp.float32),
            pltpu.VMEM((1, K), jnp.float32),
        ],
        compiler_params=pltpu.CompilerParams(
            dimension_semantics=("parallel", "arbitrary"),
        ),
        name="netvlad_fused",
    )(x, clusters, clusters2,
      bn_gamma.reshape(1, C), bn_beta.reshape(1, C),
      bn_mean.reshape(1, C), bn_var.reshape(1, C))
    return out.reshape(B, D * K)
